# initial kernel scaffold (unmeasured)
import jax
import jax.numpy as jnp
from jax import lax
from jax.experimental import pallas as pl
from jax.experimental.pallas import tpu as pltpu

CM = 512


def kernel(A, B):
    m, k = A.shape
    _, n = B.shape
    n_chunks = m // CM

    a16 = A.astype(jnp.bfloat16)
    b16 = B.astype(jnp.bfloat16)

    def body(a_ref, b_ref, out_ref, send_buf, recv_buf, send_sems, recv_sems):
        i = pl.program_id(0)
        my_x = lax.axis_index("x")
        my_y = lax.axis_index("y")
        peer = (1 - my_x, my_y)

        @pl.when(i == 0)
        def _():
            barrier_sem = pltpu.get_barrier_semaphore()
            pl.semaphore_signal(
                barrier_sem,
                inc=1,
                device_id=peer,
                device_id_type=pl.DeviceIdType.MESH,
            )
            pl.semaphore_wait(barrier_sem, 1)

        slot = lax.rem(i, 2)
        p = jnp.dot(a_ref[...], b_ref[...], preferred_element_type=jnp.float32)
        send_buf[slot] = p.astype(jnp.bfloat16)
        rdma = pltpu.make_async_remote_copy(
            src_ref=send_buf.at[slot],
            dst_ref=recv_buf.at[slot],
            send_sem=send_sems.at[slot],
            recv_sem=recv_sems.at[slot],
            device_id=peer,
            device_id_type=pl.DeviceIdType.MESH,
        )
        rdma.start()
        rdma.wait()
        out_ref[...] = p + recv_buf[slot].astype(jnp.float32)

    return pl.pallas_call(
        body,
        grid=(n_chunks,),
        in_specs=[
            pl.BlockSpec((CM, k), lambda i: (i, 0)),
            pl.BlockSpec((k, n), lambda i: (0, 0)),
        ],
        out_specs=pl.BlockSpec((CM, n), lambda i: (i, 0)),
        out_shape=jax.ShapeDtypeStruct((m, n), jnp.float32),
        scratch_shapes=[
            pltpu.VMEM((2, CM, n), jnp.bfloat16),
            pltpu.VMEM((2, CM, n), jnp.bfloat16),
            pltpu.SemaphoreType.DMA((2,)),
            pltpu.SemaphoreType.DMA((2,)),
        ],
        compiler_params=pltpu.CompilerParams(
            collective_id=0,
            dimension_semantics=("arbitrary",),
        ),
    )(a16, b16)


# baseline (device time: 544495 ns/iter reference)
import jax
import jax.numpy as jnp
from jax import lax
from jax.experimental import pallas as pl
from jax.experimental.pallas import tpu as pltpu

CM = 512


def kernel(A, B):
    m, k = A.shape
    _, n = B.shape
    n_chunks = m // CM

    a16 = A.astype(jnp.bfloat16)
    b16 = B.astype(jnp.bfloat16)

    def body(a_ref, b_ref, out_ref, send_buf, recv_buf, send_sems, recv_sems):
        i = pl.program_id(0)
        my_x = lax.axis_index("x")
        my_y = lax.axis_index("y")
        peer = (1 - my_x, my_y)

        @pl.when(i == 0)
        def _():
            barrier_sem = pltpu.get_barrier_semaphore()
            pl.semaphore_signal(
                barrier_sem,
                inc=1,
                device_id=peer,
                device_id_type=pl.DeviceIdType.MESH,
            )
            pl.semaphore_wait(barrier_sem, 1)

        slot = lax.rem(i, 2)
        p = jnp.dot(a_ref[...], b_ref[...], preferred_element_type=jnp.float32)
        send_buf[slot] = p.astype(jnp.bfloat16)
        rdma = pltpu.make_async_remote_copy(
            src_ref=send_buf.at[slot],
            dst_ref=recv_buf.at[slot],
            send_sem=send_sems.at[slot],
            recv_sem=recv_sems.at[slot],
            device_id=peer,
            device_id_type=pl.DeviceIdType.MESH,
        )
        rdma.start()
        rdma.wait()
        out_ref[...] = p + recv_buf[slot].astype(jnp.float32)

    return pl.pallas_call(
        body,
        grid=(n_chunks,),
        in_specs=[
            pl.BlockSpec((CM, k), lambda i: (i, 0)),
            pl.BlockSpec((k, n), lambda i: (0, 0)),
        ],
        out_specs=pl.BlockSpec((CM, n), lambda i: (i, 0)),
        out_shape=jax.ShapeDtypeStruct((m, n), jnp.float32),
        scratch_shapes=[
            pltpu.VMEM((2, CM, n), jnp.bfloat16),
            pltpu.VMEM((2, CM, n), jnp.bfloat16),
            pltpu.SemaphoreType.DMA((2,)),
            pltpu.SemaphoreType.DMA((2,)),
        ],
        compiler_params=pltpu.CompilerParams(
            collective_id=0,
            dimension_semantics=("arbitrary",),
            vmem_limit_bytes=64 * 1024 * 1024,
        ),
    )(a16, b16)


# device time: 459911 ns/iter; 1.1839x vs baseline; 1.1839x over previous
import jax
import jax.numpy as jnp
from jax import lax
from jax.experimental import pallas as pl
from jax.experimental.pallas import tpu as pltpu

CM = 512
N_SEND_SLOTS = 2
N_RECV_SLOTS = 4


def kernel(A, B):
    m, k = A.shape
    _, n = B.shape
    n_chunks = m // CM

    a16 = A.astype(jnp.bfloat16)
    b16 = B.astype(jnp.bfloat16)

    def body(a_ref, b_ref, out_ref, send_buf, recv_buf, send_sems, recv_sems):
        i = pl.program_id(0)
        my_x = lax.axis_index("x")
        my_y = lax.axis_index("y")
        peer = (1 - my_x, my_y)

        def rdma_for(msg):
            s = lax.rem(msg, N_SEND_SLOTS)
            r = lax.rem(msg, N_RECV_SLOTS)
            return pltpu.make_async_remote_copy(
                src_ref=send_buf.at[s],
                dst_ref=recv_buf.at[r],
                send_sem=send_sems.at[s],
                recv_sem=recv_sems.at[r],
                device_id=peer,
                device_id_type=pl.DeviceIdType.MESH,
            )

        @pl.when(i == 0)
        def _():
            barrier_sem = pltpu.get_barrier_semaphore()
            pl.semaphore_signal(
                barrier_sem,
                inc=1,
                device_id=peer,
                device_id_type=pl.DeviceIdType.MESH,
            )
            pl.semaphore_wait(barrier_sem, 1)

        @pl.when(i < n_chunks)
        def _():
            @pl.when(i >= N_SEND_SLOTS)
            def _():
                rdma_for(i - N_SEND_SLOTS).wait_send()

            s = lax.rem(i, N_SEND_SLOTS)
            p = jnp.dot(
                a_ref[...], b_ref[...], preferred_element_type=jnp.float32
            )
            send_buf[s] = p.astype(jnp.bfloat16)
            rdma_for(i).start()

        @pl.when(i >= 1)
        def _():
            rdma_for(i - 1).wait_recv()
            s = lax.rem(i - 1, N_SEND_SLOTS)
            r = lax.rem(i - 1, N_RECV_SLOTS)
            out_ref[...] = send_buf[s].astype(jnp.float32) + recv_buf[
                r
            ].astype(jnp.float32)

        @pl.when(i == n_chunks)
        def _():
            rdma_for(n_chunks - 2).wait_send()
            rdma_for(n_chunks - 1).wait_send()

    return pl.pallas_call(
        body,
        grid=(n_chunks + 1,),
        in_specs=[
            pl.BlockSpec((CM, k), lambda i: (jnp.minimum(i, m // CM - 1), 0)),
            pl.BlockSpec((k, n), lambda i: (0, 0)),
        ],
        out_specs=pl.BlockSpec((CM, n), lambda i: (jnp.maximum(i - 1, 0), 0)),
        out_shape=jax.ShapeDtypeStruct((m, n), jnp.float32),
        scratch_shapes=[
            pltpu.VMEM((N_SEND_SLOTS, CM, n), jnp.bfloat16),
            pltpu.VMEM((N_RECV_SLOTS, CM, n), jnp.bfloat16),
            pltpu.SemaphoreType.DMA((N_SEND_SLOTS,)),
            pltpu.SemaphoreType.DMA((N_RECV_SLOTS,)),
        ],
        compiler_params=pltpu.CompilerParams(
            collective_id=0,
            dimension_semantics=("arbitrary",),
            vmem_limit_bytes=64 * 1024 * 1024,
        ),
    )(a16, b16)


# device time: 453486 ns/iter; 1.2007x vs baseline; 1.0142x over previous
import jax
import jax.numpy as jnp
from jax import lax
from jax.experimental import pallas as pl
from jax.experimental.pallas import tpu as pltpu

CM = 256
N_SEND_SLOTS = 4
N_RECV_SLOTS = 8


def kernel(A, B):
    m, k = A.shape
    _, n = B.shape
    n_chunks = m // CM

    a16 = A.astype(jnp.bfloat16)
    b16 = B.astype(jnp.bfloat16)

    def body(a_ref, b_ref, out_ref, send_buf, recv_buf, send_sems, recv_sems):
        i = pl.program_id(0)
        my_x = lax.axis_index("x")
        my_y = lax.axis_index("y")
        peer = (1 - my_x, my_y)

        def rdma_for(msg):
            s = lax.rem(msg, N_SEND_SLOTS)
            r = lax.rem(msg, N_RECV_SLOTS)
            return pltpu.make_async_remote_copy(
                src_ref=send_buf.at[s],
                dst_ref=recv_buf.at[r],
                send_sem=send_sems.at[s],
                recv_sem=recv_sems.at[r],
                device_id=peer,
                device_id_type=pl.DeviceIdType.MESH,
            )

        @pl.when(i == 0)
        def _():
            barrier_sem = pltpu.get_barrier_semaphore()
            pl.semaphore_signal(
                barrier_sem,
                inc=1,
                device_id=peer,
                device_id_type=pl.DeviceIdType.MESH,
            )
            pl.semaphore_wait(barrier_sem, 1)

        @pl.when(i < n_chunks)
        def _():
            @pl.when(i >= N_SEND_SLOTS)
            def _():
                rdma_for(i - N_SEND_SLOTS).wait_send()

            s = lax.rem(i, N_SEND_SLOTS)
            p = jnp.dot(
                a_ref[...], b_ref[...], preferred_element_type=jnp.float32
            )
            send_buf[s] = p.astype(jnp.bfloat16)
            rdma_for(i).start()

        @pl.when(i >= 2)
        def _():
            rdma_for(i - 2).wait_recv()
            s = lax.rem(i - 2, N_SEND_SLOTS)
            r = lax.rem(i - 2, N_RECV_SLOTS)
            out_ref[...] = send_buf[s].astype(jnp.float32) + recv_buf[
                r
            ].astype(jnp.float32)

        @pl.when(i == n_chunks + 1)
        def _():
            for msg in range(n_chunks - N_SEND_SLOTS, n_chunks):
                rdma_for(msg).wait_send()

    return pl.pallas_call(
        body,
        grid=(n_chunks + 2,),
        in_specs=[
            pl.BlockSpec((CM, k), lambda i: (jnp.minimum(i, m // CM - 1), 0)),
            pl.BlockSpec((k, n), lambda i: (0, 0)),
        ],
        out_specs=pl.BlockSpec((CM, n), lambda i: (jnp.maximum(i - 2, 0), 0)),
        out_shape=jax.ShapeDtypeStruct((m, n), jnp.float32),
        scratch_shapes=[
            pltpu.VMEM((N_SEND_SLOTS, CM, n), jnp.bfloat16),
            pltpu.VMEM((N_RECV_SLOTS, CM, n), jnp.bfloat16),
            pltpu.SemaphoreType.DMA((N_SEND_SLOTS,)),
            pltpu.SemaphoreType.DMA((N_RECV_SLOTS,)),
        ],
        compiler_params=pltpu.CompilerParams(
            collective_id=0,
            dimension_semantics=("arbitrary",),
            vmem_limit_bytes=64 * 1024 * 1024,
        ),
    )(a16, b16)
